# baseline (speedup 1.0000x reference)
import functools

import jax
import jax.numpy as jnp
from jax import lax
from jax.experimental import pallas as pl
from jax.experimental.pallas import tpu as pltpu
from jax.experimental.pallas import tpu_sc as plsc

N = 10000
DEG = 16
NBRE = 80000
EMB = 128

NW = 32
NPAD = 10240
C = NPAD // NW
EPS = 128
DPS = EPS // DEG
J = C // DPS

_f32 = jnp.float32
_i32 = jnp.int32


DEPTH = 4


def _tree_sum(terms):
    while len(terms) > 1:
        nxt = [terms[i] + terms[i + 1] for i in range(0, len(terms) - 1, 2)]
        if len(terms) % 2:
            nxt.append(terms[-1])
        terms = nxt
    return terms[0]


def _sc_body(dev2d, brk_flat, bs_flat, vpre, ne_out, cbs_out,
             dev_v, cbs4, ie4, io4, b04, b14, nb4, rows4, ne4, *sems):
    semc = sems[0:DEPTH]
    semb = sems[DEPTH:2 * DEPTH]
    semv = sems[2 * DEPTH:3 * DEPTH]
    semsc = sems[3 * DEPTH:4 * DEPTH]
    semsn = sems[4 * DEPTH:5 * DEPTH]
    wid = lax.axis_index("s") * 2 + lax.axis_index("c")
    base = wid * C
    pltpu.sync_copy(dev2d.at[pl.ds(wid * J, J)], dev_v)

    def fire(jj, p):
        @pl.when(jj < J)
        def _():
            @pl.when(jj >= DEPTH)
            def _():
                pltpu.make_async_copy(
                    cbs4.at[p], cbs_out.at[pl.ds(0, EPS)], semsc[p]).wait()
            pltpu.async_copy(bs_flat.at[dev_v.at[jj]], cbs4.at[p], semc[p])
            for v in range(DPS):
                dv2 = dev_v[jj, pl.ds(v * 16, 16)] * 2
                ie4[p, pl.ds(v * 16, 16)] = dv2
                io4[p, pl.ds(v * 16, 16)] = dv2 + 1
            pltpu.async_copy(brk_flat.at[ie4.at[p]], b04.at[p], semb[p])
            pltpu.async_copy(brk_flat.at[io4.at[p]], b14.at[p], semb[p])

    def nb_fire_vpre(jj, p):
        @pl.when(jj < J)
        def _():
            pltpu.make_async_copy(
                brk_flat.at[ie4.at[p]], b04.at[p], semb[p]).wait()
            pltpu.make_async_copy(
                brk_flat.at[io4.at[p]], b14.at[p], semb[p]).wait()
            for v in range(DPS):
                br0 = b04[p, pl.ds(v * 16, 16)]
                br1 = b14[p, pl.ds(v * 16, 16)]
                did = jnp.full((16,), base + jj * DPS + v, _i32)
                nb4[p, pl.ds(v * 16, 16)] = jnp.where(br0 == did, br1, br0)
            pltpu.async_copy(vpre.at[nb4.at[p]], rows4.at[p], semv[p])

    def back(jj, p):
        pltpu.make_async_copy(
            bs_flat.at[dev_v.at[jj]], cbs4.at[p], semc[p]).wait()
        pltpu.async_copy(
            cbs4.at[p], cbs_out.at[pl.ds((base + jj * DPS) * DEG, EPS)],
            semsc[p])
        @pl.when(jj >= DEPTH)
        def _():
            pltpu.make_async_copy(
                ne4.at[p], ne_out.at[pl.ds(base, DPS)], semsn[p]).wait()
        pltpu.make_async_copy(vpre.at[nb4.at[p]], rows4.at[p], semv[p]).wait()

        def red_v(v, carry):
            for eb in range(EMB // 32):
                ta, tb = [], []
                for d in range(DEG):
                    w = rows4[p, v * DEG + d, pl.ds(eb * 16, 16)]
                    ta.append(lax.bitcast_convert_type(w << 16, _f32))
                    tb.append(lax.bitcast_convert_type(
                        w & jnp.int32(-65536), _f32))
                ne4[p, v, pl.ds(eb * 32, 16)] = _tree_sum(ta)
                ne4[p, v, pl.ds(eb * 32 + 16, 16)] = _tree_sum(tb)
            return carry

        lax.fori_loop(0, DPS, red_v, 0)
        pltpu.async_copy(
            ne4.at[p], ne_out.at[pl.ds(base + jj * DPS, DPS)], semsn[p])

    fire(0, 0)
    fire(1, 1)
    fire(2, 2)
    nb_fire_vpre(0, 0)
    nb_fire_vpre(1, 1)

    def body4(i, carry):
        j = 4 * i
        for k in range(4):
            nb_fire_vpre(j + k + 2, (k + 2) % 4)
            back(j + k, k)
            fire(j + k + 3, (k + 3) % 4)
        return carry

    lax.fori_loop(0, J // 4, body4, 0)
    for p in range(DEPTH):
        pltpu.make_async_copy(
            cbs4.at[p], cbs_out.at[pl.ds(0, EPS)], semsc[p]).wait()
        pltpu.make_async_copy(
            ne4.at[p], ne_out.at[pl.ds(0, DPS)], semsn[p]).wait()


@functools.cache
def _sc_gather():
    return pl.kernel(
        _sc_body,
        mesh=plsc.VectorSubcoreMesh(core_axis_name="c", subcore_axis_name="s"),
        out_type=[
            jax.ShapeDtypeStruct((NPAD, EMB), _f32),
            jax.ShapeDtypeStruct((NPAD * DEG,), _f32),
        ],
        scratch_types=[
            pltpu.VMEM((J, EPS), _i32),
            pltpu.VMEM((DEPTH, EPS), _f32),
            pltpu.VMEM((DEPTH, EPS), _i32),
            pltpu.VMEM((DEPTH, EPS), _i32),
            pltpu.VMEM((DEPTH, EPS), _i32),
            pltpu.VMEM((DEPTH, EPS), _i32),
            pltpu.VMEM((DEPTH, EPS), _i32),
            pltpu.VMEM((DEPTH, EPS, EMB // 2), _i32),
            pltpu.VMEM((DEPTH, DPS, EMB), _f32),
        ] + [pltpu.SemaphoreType.DMA] * (5 * DEPTH),
        compiler_params=pltpu.CompilerParams(use_tc_tiling_on_sc=False),
    )


BLK = 256


def _tc_body(cbs_ref, ne_ref, ps_ref, W0_ref, W3_ref, W5_ref,
             w1_ref, w2_ref, w4_ref, b0_ref, b1_ref, b2_ref, b3_ref,
             b4_ref, b5_ref, wcb_ref, out_ref):
    cbs = cbs_ref[...]
    w4 = w4_ref[...]
    b4 = b4_ref[...]
    be = jnp.tanh(cbs[:, 0:1] * w4 + b4)
    for d in range(1, DEG):
        be = be + jnp.tanh(cbs[:, d:d + 1] * w4 + b4)
    dn = (((1,), (1,)), ((), ()))
    brk = jnp.tanh(lax.dot_general(be, W3_ref[...], dn,
                                   preferred_element_type=_f32) + b3_ref[...])
    tmp = jnp.sum(cbs, axis=1, keepdims=True)
    tmp_emb = jnp.tanh(tmp * w2_ref[...] + b2_ref[...])
    ps = ps_ref[...]
    pe = 3.0 * tmp_emb
    for p in range(3):
        pe = pe + jnp.tanh(ps[:, p:p + 1] * w1_ref[...] + b1_ref[...])
    pro = jnp.tanh(lax.dot_general(pe, W0_ref[...], dn,
                                   preferred_element_type=_f32) + b0_ref[...])
    nei = jnp.tanh(lax.dot_general(ne_ref[...], W5_ref[...], dn,
                                   preferred_element_type=_f32) + b5_ref[...])
    wcb = wcb_ref[...]
    out_ref[...] = jnp.tanh(pro * wcb[0:1, :] + brk * wcb[1:2, :]
                            + nei * wcb[2:3, :] + wcb[3:4, :])


def _row_spec(width):
    return pl.BlockSpec((BLK, width), lambda i: (i, 0))


def _w_spec(rows, cols):
    return pl.BlockSpec((rows, cols), lambda i: (0, 0))


_tc_dense = pl.pallas_call(
    _tc_body,
    grid=(NPAD // BLK,),
    in_specs=[
        _row_spec(DEG),
        _row_spec(EMB),
        _row_spec(3),
        _w_spec(EMB, EMB),
        _w_spec(EMB, EMB),
        _w_spec(EMB, EMB),
        _w_spec(1, EMB),
        _w_spec(1, EMB),
        _w_spec(1, EMB),
        _w_spec(1, EMB),
        _w_spec(1, EMB),
        _w_spec(1, EMB),
        _w_spec(1, EMB),
        _w_spec(1, EMB),
        _w_spec(1, EMB),
        _w_spec(4, EMB),
    ],
    out_specs=_row_spec(EMB),
    out_shape=jax.ShapeDtypeStruct((NPAD, EMB), _f32),
)


def kernel(V_pre, devices, breakers, protector_sate, breaker_state,
           W0, b0, W1, b1, W2, b2, W3, b3, W4, b4, W5, b5, Wc, bc):
    dev = jnp.pad(devices.astype(_i32), ((0, NPAD - N), (0, 0)))
    dev2d = dev.reshape(NPAD * DEG // EPS, EPS)
    vperm = jnp.transpose(V_pre.reshape(N, EMB // 32, 2, 16),
                          (0, 1, 3, 2)).reshape(N, EMB)
    vpre_b = jax.lax.bitcast_convert_type(
        vperm.astype(jnp.bfloat16).reshape(N, EMB // 2, 2), _i32)
    ne_pad, cbs_flat = _sc_gather()(dev2d, breakers.astype(_i32).reshape(-1),
                                    breaker_state, vpre_b)
    cbs_pad = cbs_flat.reshape(NPAD, DEG)
    ps_pad = jnp.pad(protector_sate, ((0, NPAD - N), (0, 0)))
    row = lambda v: v.reshape(1, EMB)
    wcb = jnp.concatenate([
        jnp.full((1, EMB), Wc[0]), jnp.full((1, EMB), Wc[1]),
        jnp.full((1, EMB), Wc[2]), jnp.full((1, EMB), bc[0]),
    ], axis=0)
    out_pad = _tc_dense(cbs_pad, ne_pad, ps_pad, W0, W3, W5,
                        row(W1[:, 0]), row(W2[:, 0]), row(W4[:, 0]),
                        row(b0), row(b1), row(b2), row(b3), row(b4), row(b5),
                        wcb)
    return out_pad[:N]


# speedup vs baseline: 1.0464x; 1.0464x over previous
import functools

import jax
import jax.numpy as jnp
from jax import lax
from jax.experimental import pallas as pl
from jax.experimental.pallas import tpu as pltpu
from jax.experimental.pallas import tpu_sc as plsc

N = 10000
DEG = 16
NBRE = 80000
EMB = 128

NW = 32
NPAD = 10240
EPS = 128
DPS = EPS // DEG
C0 = 416
C1 = 224
J0 = C0 // DPS
J1 = C1 // DPS

_f32 = jnp.float32
_i32 = jnp.int32


DEPTH = 4


def _tree_sum(terms):
    while len(terms) > 1:
        nxt = [terms[i] + terms[i + 1] for i in range(0, len(terms) - 1, 2)]
        if len(terms) % 2:
            nxt.append(terms[-1])
        terms = nxt
    return terms[0]


def _sc_body(dev2d, brk_flat, bs_flat, vpre, ne_out, cbs_out,
             dev_v, cbs4, ie4, io4, b04, b14, nb4, rows4, ne4, *sems):
    semc = sems[0:DEPTH]
    semb = sems[DEPTH:2 * DEPTH]
    semv = sems[2 * DEPTH:3 * DEPTH]
    semsc = sems[3 * DEPTH:4 * DEPTH]
    semsn = sems[4 * DEPTH:5 * DEPTH]
    s = lax.axis_index("s")
    c = lax.axis_index("c")
    base = jnp.where(c == 0, s * C0, 16 * C0 + s * C1)
    jcnt = jnp.where(c == 0, J0, J1)
    pltpu.sync_copy(dev2d.at[pl.ds(base // DPS, J0)], dev_v)

    def fire(jj, p):
        @pl.when(jj < jcnt)
        def _():
            @pl.when(jj >= DEPTH)
            def _():
                pltpu.make_async_copy(
                    cbs4.at[p], cbs_out.at[pl.ds(0, EPS)], semsc[p]).wait()
            pltpu.async_copy(bs_flat.at[dev_v.at[jj]], cbs4.at[p], semc[p])
            for v in range(DPS):
                dv2 = dev_v[jj, pl.ds(v * 16, 16)] * 2
                ie4[p, pl.ds(v * 16, 16)] = dv2
                io4[p, pl.ds(v * 16, 16)] = dv2 + 1
            pltpu.async_copy(brk_flat.at[ie4.at[p]], b04.at[p], semb[p])
            pltpu.async_copy(brk_flat.at[io4.at[p]], b14.at[p], semb[p])

    def nb_fire_vpre(jj, p):
        @pl.when(jj < jcnt)
        def _():
            pltpu.make_async_copy(
                brk_flat.at[ie4.at[p]], b04.at[p], semb[p]).wait()
            pltpu.make_async_copy(
                brk_flat.at[io4.at[p]], b14.at[p], semb[p]).wait()
            for v in range(DPS):
                br0 = b04[p, pl.ds(v * 16, 16)]
                br1 = b14[p, pl.ds(v * 16, 16)]
                did = jnp.full((16,), base + jj * DPS + v, _i32)
                nb4[p, pl.ds(v * 16, 16)] = jnp.where(br0 == did, br1, br0)
            pltpu.async_copy(vpre.at[nb4.at[p]], rows4.at[p], semv[p])

    def back(jj, p):
        pltpu.make_async_copy(
            bs_flat.at[dev_v.at[jj]], cbs4.at[p], semc[p]).wait()
        pltpu.async_copy(
            cbs4.at[p], cbs_out.at[pl.ds((base + jj * DPS) * DEG, EPS)],
            semsc[p])
        @pl.when(jj >= DEPTH)
        def _():
            pltpu.make_async_copy(
                ne4.at[p], ne_out.at[pl.ds(base, DPS)], semsn[p]).wait()
        pltpu.make_async_copy(vpre.at[nb4.at[p]], rows4.at[p], semv[p]).wait()

        def red_v(v, carry):
            for eb in range(EMB // 32):
                ta, tb = [], []
                for d in range(DEG):
                    w = rows4[p, v * DEG + d, pl.ds(eb * 16, 16)]
                    ta.append(lax.bitcast_convert_type(w << 16, _f32))
                    tb.append(lax.bitcast_convert_type(
                        w & jnp.int32(-65536), _f32))
                ne4[p, v, pl.ds(eb * 32, 16)] = _tree_sum(ta)
                ne4[p, v, pl.ds(eb * 32 + 16, 16)] = _tree_sum(tb)
            return carry

        lax.fori_loop(0, DPS, red_v, 0)
        pltpu.async_copy(
            ne4.at[p], ne_out.at[pl.ds(base + jj * DPS, DPS)], semsn[p])

    fire(0, 0)
    fire(1, 1)
    fire(2, 2)
    nb_fire_vpre(0, 0)
    nb_fire_vpre(1, 1)

    def body4(i, carry):
        j = 4 * i
        for k in range(4):
            nb_fire_vpre(j + k + 2, (k + 2) % 4)
            back(j + k, k)
            fire(j + k + 3, (k + 3) % 4)
        return carry

    lax.fori_loop(0, jcnt // 4, body4, 0)
    for p in range(DEPTH):
        pltpu.make_async_copy(
            cbs4.at[p], cbs_out.at[pl.ds(0, EPS)], semsc[p]).wait()
        pltpu.make_async_copy(
            ne4.at[p], ne_out.at[pl.ds(0, DPS)], semsn[p]).wait()


@functools.cache
def _sc_gather():
    return pl.kernel(
        _sc_body,
        mesh=plsc.VectorSubcoreMesh(core_axis_name="c", subcore_axis_name="s"),
        out_type=[
            jax.ShapeDtypeStruct((NPAD, EMB), _f32),
            jax.ShapeDtypeStruct((NPAD * DEG,), _f32),
        ],
        scratch_types=[
            pltpu.VMEM((J0, EPS), _i32),
            pltpu.VMEM((DEPTH, EPS), _f32),
            pltpu.VMEM((DEPTH, EPS), _i32),
            pltpu.VMEM((DEPTH, EPS), _i32),
            pltpu.VMEM((DEPTH, EPS), _i32),
            pltpu.VMEM((DEPTH, EPS), _i32),
            pltpu.VMEM((DEPTH, EPS), _i32),
            pltpu.VMEM((DEPTH, EPS, EMB // 2), _i32),
            pltpu.VMEM((DEPTH, DPS, EMB), _f32),
        ] + [pltpu.SemaphoreType.DMA] * (5 * DEPTH),
        compiler_params=pltpu.CompilerParams(use_tc_tiling_on_sc=False),
    )


BLK = 256


def _tc_body(cbs_ref, ne_ref, ps_ref, W0_ref, W3_ref, W5_ref,
             w1_ref, w2_ref, w4_ref, b0_ref, b1_ref, b2_ref, b3_ref,
             b4_ref, b5_ref, wcb_ref, out_ref):
    cbs = cbs_ref[...]
    w4 = w4_ref[...]
    b4 = b4_ref[...]
    be = jnp.tanh(cbs[:, 0:1] * w4 + b4)
    for d in range(1, DEG):
        be = be + jnp.tanh(cbs[:, d:d + 1] * w4 + b4)
    dn = (((1,), (1,)), ((), ()))
    brk = jnp.tanh(lax.dot_general(be, W3_ref[...], dn,
                                   preferred_element_type=_f32) + b3_ref[...])
    tmp = jnp.sum(cbs, axis=1, keepdims=True)
    tmp_emb = jnp.tanh(tmp * w2_ref[...] + b2_ref[...])
    ps = ps_ref[...]
    pe = 3.0 * tmp_emb
    for p in range(3):
        pe = pe + jnp.tanh(ps[:, p:p + 1] * w1_ref[...] + b1_ref[...])
    pro = jnp.tanh(lax.dot_general(pe, W0_ref[...], dn,
                                   preferred_element_type=_f32) + b0_ref[...])
    nei = jnp.tanh(lax.dot_general(ne_ref[...], W5_ref[...], dn,
                                   preferred_element_type=_f32) + b5_ref[...])
    wcb = wcb_ref[...]
    out_ref[...] = jnp.tanh(pro * wcb[0:1, :] + brk * wcb[1:2, :]
                            + nei * wcb[2:3, :] + wcb[3:4, :])


def _row_spec(width):
    return pl.BlockSpec((BLK, width), lambda i: (i, 0))


def _w_spec(rows, cols):
    return pl.BlockSpec((rows, cols), lambda i: (0, 0))


_tc_dense = pl.pallas_call(
    _tc_body,
    grid=(NPAD // BLK,),
    in_specs=[
        _row_spec(DEG),
        _row_spec(EMB),
        _row_spec(3),
        _w_spec(EMB, EMB),
        _w_spec(EMB, EMB),
        _w_spec(EMB, EMB),
        _w_spec(1, EMB),
        _w_spec(1, EMB),
        _w_spec(1, EMB),
        _w_spec(1, EMB),
        _w_spec(1, EMB),
        _w_spec(1, EMB),
        _w_spec(1, EMB),
        _w_spec(1, EMB),
        _w_spec(1, EMB),
        _w_spec(4, EMB),
    ],
    out_specs=_row_spec(EMB),
    out_shape=jax.ShapeDtypeStruct((NPAD, EMB), _f32),
)


def kernel(V_pre, devices, breakers, protector_sate, breaker_state,
           W0, b0, W1, b1, W2, b2, W3, b3, W4, b4, W5, b5, Wc, bc):
    dev = jnp.pad(devices.astype(_i32), ((0, NPAD - N), (0, 0)))
    dev2d = dev.reshape(NPAD * DEG // EPS, EPS)
    dev2d = jnp.pad(dev2d, ((0, J0), (0, 0)))
    vperm = jnp.transpose(V_pre.reshape(N, EMB // 32, 2, 16),
                          (0, 1, 3, 2)).reshape(N, EMB)
    vpre_b = jax.lax.bitcast_convert_type(
        vperm.astype(jnp.bfloat16).reshape(N, EMB // 2, 2), _i32)
    ne_pad, cbs_flat = _sc_gather()(dev2d, breakers.astype(_i32).reshape(-1),
                                    breaker_state, vpre_b)
    cbs_pad = cbs_flat.reshape(NPAD, DEG)
    ps_pad = jnp.pad(protector_sate, ((0, NPAD - N), (0, 0)))
    row = lambda v: v.reshape(1, EMB)
    wcb = jnp.concatenate([
        jnp.full((1, EMB), Wc[0]), jnp.full((1, EMB), Wc[1]),
        jnp.full((1, EMB), Wc[2]), jnp.full((1, EMB), bc[0]),
    ], axis=0)
    out_pad = _tc_dense(cbs_pad, ne_pad, ps_pad, W0, W3, W5,
                        row(W1[:, 0]), row(W2[:, 0]), row(W4[:, 0]),
                        row(b0), row(b1), row(b2), row(b3), row(b4), row(b5),
                        wcb)
    return out_pad[:N]


# speedup vs baseline: 1.0847x; 1.0365x over previous
import functools

import jax
import jax.numpy as jnp
from jax import lax
from jax.experimental import pallas as pl
from jax.experimental.pallas import tpu as pltpu
from jax.experimental.pallas import tpu_sc as plsc

N = 10000
DEG = 16
NBRE = 80000
EMB = 128

NW = 32
NPAD = 10240
EPS = 128
DPS = EPS // DEG
C0 = 448
C1 = 192
J0 = C0 // DPS
J1 = C1 // DPS

_f32 = jnp.float32
_i32 = jnp.int32


DEPTH = 4


def _tree_sum(terms):
    while len(terms) > 1:
        nxt = [terms[i] + terms[i + 1] for i in range(0, len(terms) - 1, 2)]
        if len(terms) % 2:
            nxt.append(terms[-1])
        terms = nxt
    return terms[0]


def _sc_body(dev2d, brk_flat, bs_flat, vpre, ne_out, cbs_out,
             dev_v, cbs4, ie4, io4, b04, b14, nb4, rows4, ne4, *sems):
    semc = sems[0:DEPTH]
    semb = sems[DEPTH:2 * DEPTH]
    semv = sems[2 * DEPTH:3 * DEPTH]
    semsc = sems[3 * DEPTH:4 * DEPTH]
    semsn = sems[4 * DEPTH:5 * DEPTH]
    s = lax.axis_index("s")
    c = lax.axis_index("c")
    base = jnp.where(c == 0, s * C0, 16 * C0 + s * C1)
    jcnt = jnp.where(c == 0, J0, J1)
    pltpu.sync_copy(dev2d.at[pl.ds(base // DPS, J0)], dev_v)

    def fire(jj, p):
        @pl.when(jj < jcnt)
        def _():
            @pl.when(jj >= DEPTH)
            def _():
                pltpu.make_async_copy(
                    cbs4.at[p], cbs_out.at[pl.ds(0, EPS)], semsc[p]).wait()
            pltpu.async_copy(bs_flat.at[dev_v.at[jj]], cbs4.at[p], semc[p])
            for v in range(DPS):
                dv2 = dev_v[jj, pl.ds(v * 16, 16)] * 2
                ie4[p, pl.ds(v * 16, 16)] = dv2
                io4[p, pl.ds(v * 16, 16)] = dv2 + 1
            pltpu.async_copy(brk_flat.at[ie4.at[p]], b04.at[p], semb[p])
            pltpu.async_copy(brk_flat.at[io4.at[p]], b14.at[p], semb[p])

    def nb_fire_vpre(jj, p):
        @pl.when(jj < jcnt)
        def _():
            pltpu.make_async_copy(
                brk_flat.at[ie4.at[p]], b04.at[p], semb[p]).wait()
            pltpu.make_async_copy(
                brk_flat.at[io4.at[p]], b14.at[p], semb[p]).wait()
            for v in range(DPS):
                br0 = b04[p, pl.ds(v * 16, 16)]
                br1 = b14[p, pl.ds(v * 16, 16)]
                did = jnp.full((16,), base + jj * DPS + v, _i32)
                nb4[p, pl.ds(v * 16, 16)] = jnp.where(br0 == did, br1, br0)
            pltpu.async_copy(vpre.at[nb4.at[p]], rows4.at[p], semv[p])

    def back(jj, p):
        pltpu.make_async_copy(
            bs_flat.at[dev_v.at[jj]], cbs4.at[p], semc[p]).wait()
        pltpu.async_copy(
            cbs4.at[p], cbs_out.at[pl.ds((base + jj * DPS) * DEG, EPS)],
            semsc[p])
        @pl.when(jj >= DEPTH)
        def _():
            pltpu.make_async_copy(
                ne4.at[p], ne_out.at[pl.ds(base, DPS)], semsn[p]).wait()
        pltpu.make_async_copy(vpre.at[nb4.at[p]], rows4.at[p], semv[p]).wait()

        def red_v(v, carry):
            for eb in range(EMB // 32):
                ta, tb = [], []
                for d in range(DEG):
                    w = rows4[p, v * DEG + d, pl.ds(eb * 16, 16)]
                    ta.append(lax.bitcast_convert_type(w << 16, _f32))
                    tb.append(lax.bitcast_convert_type(
                        w & jnp.int32(-65536), _f32))
                ne4[p, v, pl.ds(eb * 32, 16)] = _tree_sum(ta)
                ne4[p, v, pl.ds(eb * 32 + 16, 16)] = _tree_sum(tb)
            return carry

        lax.fori_loop(0, DPS, red_v, 0)
        pltpu.async_copy(
            ne4.at[p], ne_out.at[pl.ds(base + jj * DPS, DPS)], semsn[p])

    fire(0, 0)
    fire(1, 1)
    fire(2, 2)
    nb_fire_vpre(0, 0)
    nb_fire_vpre(1, 1)

    def body4(i, carry):
        j = 4 * i
        for k in range(4):
            nb_fire_vpre(j + k + 2, (k + 2) % 4)
            back(j + k, k)
            fire(j + k + 3, (k + 3) % 4)
        return carry

    lax.fori_loop(0, jcnt // 4, body4, 0)
    for p in range(DEPTH):
        pltpu.make_async_copy(
            cbs4.at[p], cbs_out.at[pl.ds(0, EPS)], semsc[p]).wait()
        pltpu.make_async_copy(
            ne4.at[p], ne_out.at[pl.ds(0, DPS)], semsn[p]).wait()


@functools.cache
def _sc_gather():
    return pl.kernel(
        _sc_body,
        mesh=plsc.VectorSubcoreMesh(core_axis_name="c", subcore_axis_name="s"),
        out_type=[
            jax.ShapeDtypeStruct((NPAD, EMB), _f32),
            jax.ShapeDtypeStruct((NPAD * DEG,), _f32),
        ],
        scratch_types=[
            pltpu.VMEM((J0, EPS), _i32),
            pltpu.VMEM((DEPTH, EPS), _f32),
            pltpu.VMEM((DEPTH, EPS), _i32),
            pltpu.VMEM((DEPTH, EPS), _i32),
            pltpu.VMEM((DEPTH, EPS), _i32),
            pltpu.VMEM((DEPTH, EPS), _i32),
            pltpu.VMEM((DEPTH, EPS), _i32),
            pltpu.VMEM((DEPTH, EPS, EMB // 2), _i32),
            pltpu.VMEM((DEPTH, DPS, EMB), _f32),
        ] + [pltpu.SemaphoreType.DMA] * (5 * DEPTH),
        compiler_params=pltpu.CompilerParams(use_tc_tiling_on_sc=False),
    )


BLK = 256


def _tc_body(cbs_ref, ne_ref, ps_ref, W0_ref, W3_ref, W5_ref,
             w1_ref, w2_ref, w4_ref, b0_ref, b1_ref, b2_ref, b3_ref,
             b4_ref, b5_ref, wcb_ref, out_ref):
    cbs = cbs_ref[...]
    w4 = w4_ref[...]
    b4 = b4_ref[...]
    be = jnp.tanh(cbs[:, 0:1] * w4 + b4)
    for d in range(1, DEG):
        be = be + jnp.tanh(cbs[:, d:d + 1] * w4 + b4)
    dn = (((1,), (1,)), ((), ()))
    brk = jnp.tanh(lax.dot_general(be, W3_ref[...], dn,
                                   preferred_element_type=_f32) + b3_ref[...])
    tmp = jnp.sum(cbs, axis=1, keepdims=True)
    tmp_emb = jnp.tanh(tmp * w2_ref[...] + b2_ref[...])
    ps = ps_ref[...]
    pe = 3.0 * tmp_emb
    for p in range(3):
        pe = pe + jnp.tanh(ps[:, p:p + 1] * w1_ref[...] + b1_ref[...])
    pro = jnp.tanh(lax.dot_general(pe, W0_ref[...], dn,
                                   preferred_element_type=_f32) + b0_ref[...])
    nei = jnp.tanh(lax.dot_general(ne_ref[...], W5_ref[...], dn,
                                   preferred_element_type=_f32) + b5_ref[...])
    wcb = wcb_ref[...]
    out_ref[...] = jnp.tanh(pro * wcb[0:1, :] + brk * wcb[1:2, :]
                            + nei * wcb[2:3, :] + wcb[3:4, :])


def _row_spec(width):
    return pl.BlockSpec((BLK, width), lambda i: (i, 0))


def _w_spec(rows, cols):
    return pl.BlockSpec((rows, cols), lambda i: (0, 0))


_tc_dense = pl.pallas_call(
    _tc_body,
    grid=(NPAD // BLK,),
    in_specs=[
        _row_spec(DEG),
        _row_spec(EMB),
        _row_spec(3),
        _w_spec(EMB, EMB),
        _w_spec(EMB, EMB),
        _w_spec(EMB, EMB),
        _w_spec(1, EMB),
        _w_spec(1, EMB),
        _w_spec(1, EMB),
        _w_spec(1, EMB),
        _w_spec(1, EMB),
        _w_spec(1, EMB),
        _w_spec(1, EMB),
        _w_spec(1, EMB),
        _w_spec(1, EMB),
        _w_spec(4, EMB),
    ],
    out_specs=_row_spec(EMB),
    out_shape=jax.ShapeDtypeStruct((NPAD, EMB), _f32),
)


def kernel(V_pre, devices, breakers, protector_sate, breaker_state,
           W0, b0, W1, b1, W2, b2, W3, b3, W4, b4, W5, b5, Wc, bc):
    dev = jnp.pad(devices.astype(_i32), ((0, NPAD - N), (0, 0)))
    dev2d = dev.reshape(NPAD * DEG // EPS, EPS)
    dev2d = jnp.pad(dev2d, ((0, J0), (0, 0)))
    vperm = jnp.transpose(V_pre.reshape(N, EMB // 32, 2, 16),
                          (0, 1, 3, 2)).reshape(N, EMB)
    vpre_b = jax.lax.bitcast_convert_type(
        vperm.astype(jnp.bfloat16).reshape(N, EMB // 2, 2), _i32)
    ne_pad, cbs_flat = _sc_gather()(dev2d, breakers.astype(_i32).reshape(-1),
                                    breaker_state, vpre_b)
    cbs_pad = cbs_flat.reshape(NPAD, DEG)
    ps_pad = jnp.pad(protector_sate, ((0, NPAD - N), (0, 0)))
    row = lambda v: v.reshape(1, EMB)
    wcb = jnp.concatenate([
        jnp.full((1, EMB), Wc[0]), jnp.full((1, EMB), Wc[1]),
        jnp.full((1, EMB), Wc[2]), jnp.full((1, EMB), bc[0]),
    ], axis=0)
    out_pad = _tc_dense(cbs_pad, ne_pad, ps_pad, W0, W3, W5,
                        row(W1[:, 0]), row(W2[:, 0]), row(W4[:, 0]),
                        row(b0), row(b1), row(b2), row(b3), row(b4), row(b5),
                        wcb)
    return out_pad[:N]


# speedup vs baseline: 1.1160x; 1.0289x over previous
import functools

import jax
import jax.numpy as jnp
from jax import lax
from jax.experimental import pallas as pl
from jax.experimental.pallas import tpu as pltpu
from jax.experimental.pallas import tpu_sc as plsc

N = 10000
DEG = 16
NBRE = 80000
EMB = 128

NW = 32
NPAD = 10240
EPS = 128
DPS = EPS // DEG
C0 = 448
C1 = 192
J0 = C0 // DPS
J1 = C1 // DPS

_f32 = jnp.float32
_i32 = jnp.int32


DEPTH = 4


def _tree_sum(terms):
    while len(terms) > 1:
        nxt = [terms[i] + terms[i + 1] for i in range(0, len(terms) - 1, 2)]
        if len(terms) % 2:
            nxt.append(terms[-1])
        terms = nxt
    return terms[0]


def _sc_body(dev2d, brk_flat, bs_flat, vpre, ne_out, cbs_out,
             dev_v, cbs4, ie4, io4, b04, b14, nb4, rows4, ne4, *sems):
    semc = sems[0:DEPTH]
    semb = sems[DEPTH:2 * DEPTH]
    semv = sems[2 * DEPTH:3 * DEPTH]
    semsc = sems[3 * DEPTH:4 * DEPTH]
    semsn = sems[4 * DEPTH:5 * DEPTH]
    s = lax.axis_index("s")
    c = lax.axis_index("c")
    base = jnp.where(c == 0, s * C0, 16 * C0 + s * C1)
    jcnt = jnp.where(c == 0, J0, J1)
    pltpu.sync_copy(dev2d.at[pl.ds(base // DPS, J0)], dev_v)

    def fire(jj, p):
        @pl.when(jj < jcnt)
        def _():
            @pl.when(jj >= DEPTH)
            def _():
                pltpu.make_async_copy(
                    cbs4.at[p], cbs_out.at[pl.ds(0, EPS)], semsc[p]).wait()
            pltpu.async_copy(bs_flat.at[dev_v.at[jj]], cbs4.at[p], semc[p])
            for v in range(DPS):
                dv2 = dev_v[jj, pl.ds(v * 16, 16)] * 2
                ie4[p, pl.ds(v * 16, 16)] = dv2
                io4[p, pl.ds(v * 16, 16)] = dv2 + 1
            pltpu.async_copy(brk_flat.at[ie4.at[p]], b04.at[p], semb[p])
            pltpu.async_copy(brk_flat.at[io4.at[p]], b14.at[p], semb[p])

    def nb_fire_vpre(jj, p):
        @pl.when(jj < jcnt)
        def _():
            pltpu.make_async_copy(
                brk_flat.at[ie4.at[p]], b04.at[p], semb[p]).wait()
            pltpu.make_async_copy(
                brk_flat.at[io4.at[p]], b14.at[p], semb[p]).wait()
            for v in range(DPS):
                br0 = b04[p, pl.ds(v * 16, 16)]
                br1 = b14[p, pl.ds(v * 16, 16)]
                did = jnp.full((16,), base + jj * DPS + v, _i32)
                nb4[p, pl.ds(v * 16, 16)] = jnp.where(br0 == did, br1, br0)
            pltpu.async_copy(vpre.at[nb4.at[p]], rows4.at[p], semv[p])

    def back(jj, p):
        pltpu.make_async_copy(
            bs_flat.at[dev_v.at[jj]], cbs4.at[p], semc[p]).wait()
        pltpu.async_copy(
            cbs4.at[p], cbs_out.at[pl.ds((base + jj * DPS) * DEG, EPS)],
            semsc[p])
        @pl.when(jj >= DEPTH)
        def _():
            pltpu.make_async_copy(
                ne4.at[p], ne_out.at[pl.ds(base, DPS)], semsn[p]).wait()
        pltpu.make_async_copy(vpre.at[nb4.at[p]], rows4.at[p], semv[p]).wait()

        def red_v(v, carry):
            for eb in range(EMB // 32):
                ta, tb = [], []
                for d in range(DEG):
                    w = rows4[p, v * DEG + d, pl.ds(eb * 16, 16)]
                    ta.append(lax.bitcast_convert_type(w << 16, _f32))
                    tb.append(lax.bitcast_convert_type(
                        w & jnp.int32(-65536), _f32))
                ne4[p, v, pl.ds(eb * 32, 16)] = _tree_sum(ta)
                ne4[p, v, pl.ds(eb * 32 + 16, 16)] = _tree_sum(tb)
            return carry

        lax.fori_loop(0, DPS, red_v, 0)
        pltpu.async_copy(
            ne4.at[p], ne_out.at[pl.ds(base + jj * DPS, DPS)], semsn[p])

    fire(0, 0)
    fire(1, 1)
    fire(2, 2)
    nb_fire_vpre(0, 0)
    nb_fire_vpre(1, 1)

    def body4(i, carry):
        j = 4 * i
        for k in range(4):
            nb_fire_vpre(j + k + 2, (k + 2) % 4)
            back(j + k, k)
            fire(j + k + 3, (k + 3) % 4)
        return carry

    lax.fori_loop(0, jcnt // 4, body4, 0)
    for p in range(DEPTH):
        pltpu.make_async_copy(
            cbs4.at[p], cbs_out.at[pl.ds(0, EPS)], semsc[p]).wait()
        pltpu.make_async_copy(
            ne4.at[p], ne_out.at[pl.ds(0, DPS)], semsn[p]).wait()


@functools.cache
def _sc_gather():
    return pl.kernel(
        _sc_body,
        mesh=plsc.VectorSubcoreMesh(core_axis_name="c", subcore_axis_name="s"),
        out_type=[
            jax.ShapeDtypeStruct((NPAD, EMB), _f32),
            jax.ShapeDtypeStruct((NPAD * DEG,), _f32),
        ],
        scratch_types=[
            pltpu.VMEM((J0, EPS), _i32),
            pltpu.VMEM((DEPTH, EPS), _f32),
            pltpu.VMEM((DEPTH, EPS), _i32),
            pltpu.VMEM((DEPTH, EPS), _i32),
            pltpu.VMEM((DEPTH, EPS), _i32),
            pltpu.VMEM((DEPTH, EPS), _i32),
            pltpu.VMEM((DEPTH, EPS), _i32),
            pltpu.VMEM((DEPTH, EPS, EMB // 2), _i32),
            pltpu.VMEM((DEPTH, DPS, EMB), _f32),
        ] + [pltpu.SemaphoreType.DMA] * (5 * DEPTH),
        compiler_params=pltpu.CompilerParams(use_tc_tiling_on_sc=False),
    )


BLK = 512


def _tc_body(cbs_ref, ne_ref, ps_ref, W0_ref, W3_ref, W5_ref,
             w1_ref, w2_ref, w4_ref, b0_ref, b1_ref, b2_ref, b3_ref,
             b4_ref, b5_ref, wcb_ref, out_ref):
    cbs = cbs_ref[...]
    w4 = w4_ref[...]
    b4 = b4_ref[...]
    be = jnp.tanh(cbs[:, 0:1] * w4 + b4)
    for d in range(1, DEG):
        be = be + jnp.tanh(cbs[:, d:d + 1] * w4 + b4)
    dn = (((1,), (1,)), ((), ()))
    brk = jnp.tanh(lax.dot_general(be, W3_ref[...], dn,
                                   preferred_element_type=_f32) + b3_ref[...])
    tmp = jnp.sum(cbs, axis=1, keepdims=True)
    tmp_emb = jnp.tanh(tmp * w2_ref[...] + b2_ref[...])
    ps = ps_ref[...]
    pe = 3.0 * tmp_emb
    for p in range(3):
        pe = pe + jnp.tanh(ps[:, p:p + 1] * w1_ref[...] + b1_ref[...])
    pro = jnp.tanh(lax.dot_general(pe, W0_ref[...], dn,
                                   preferred_element_type=_f32) + b0_ref[...])
    nei = jnp.tanh(lax.dot_general(ne_ref[...], W5_ref[...], dn,
                                   preferred_element_type=_f32) + b5_ref[...])
    wcb = wcb_ref[...]
    out_ref[...] = jnp.tanh(pro * wcb[0:1, :] + brk * wcb[1:2, :]
                            + nei * wcb[2:3, :] + wcb[3:4, :])


def _row_spec(width):
    return pl.BlockSpec((BLK, width), lambda i: (i, 0))


def _w_spec(rows, cols):
    return pl.BlockSpec((rows, cols), lambda i: (0, 0))


_tc_dense = pl.pallas_call(
    _tc_body,
    grid=(NPAD // BLK,),
    in_specs=[
        _row_spec(DEG),
        _row_spec(EMB),
        _row_spec(3),
        _w_spec(EMB, EMB),
        _w_spec(EMB, EMB),
        _w_spec(EMB, EMB),
        _w_spec(1, EMB),
        _w_spec(1, EMB),
        _w_spec(1, EMB),
        _w_spec(1, EMB),
        _w_spec(1, EMB),
        _w_spec(1, EMB),
        _w_spec(1, EMB),
        _w_spec(1, EMB),
        _w_spec(1, EMB),
        _w_spec(4, EMB),
    ],
    out_specs=_row_spec(EMB),
    out_shape=jax.ShapeDtypeStruct((NPAD, EMB), _f32),
)


def kernel(V_pre, devices, breakers, protector_sate, breaker_state,
           W0, b0, W1, b1, W2, b2, W3, b3, W4, b4, W5, b5, Wc, bc):
    dev = jnp.pad(devices.astype(_i32), ((0, NPAD - N), (0, 0)))
    dev2d = dev.reshape(NPAD * DEG // EPS, EPS)
    dev2d = jnp.pad(dev2d, ((0, J0), (0, 0)))
    vperm = jnp.transpose(V_pre.reshape(N, EMB // 32, 2, 16),
                          (0, 1, 3, 2)).reshape(N, EMB)
    vpre_b = jax.lax.bitcast_convert_type(
        vperm.astype(jnp.bfloat16).reshape(N, EMB // 2, 2), _i32)
    ne_pad, cbs_flat = _sc_gather()(dev2d, breakers.astype(_i32).reshape(-1),
                                    breaker_state, vpre_b)
    cbs_pad = cbs_flat.reshape(NPAD, DEG)
    ps_pad = jnp.pad(protector_sate, ((0, NPAD - N), (0, 0)))
    row = lambda v: v.reshape(1, EMB)
    wcb = jnp.concatenate([
        jnp.full((1, EMB), Wc[0]), jnp.full((1, EMB), Wc[1]),
        jnp.full((1, EMB), Wc[2]), jnp.full((1, EMB), bc[0]),
    ], axis=0)
    out_pad = _tc_dense(cbs_pad, ne_pad, ps_pad, W0, W3, W5,
                        row(W1[:, 0]), row(W2[:, 0]), row(W4[:, 0]),
                        row(b0), row(b1), row(b2), row(b3), row(b4), row(b5),
                        wcb)
    return out_pad[:N]


# speedup vs baseline: 1.1257x; 1.0087x over previous
import functools

import jax
import jax.numpy as jnp
from jax import lax
from jax.experimental import pallas as pl
from jax.experimental.pallas import tpu as pltpu
from jax.experimental.pallas import tpu_sc as plsc

N = 10000
DEG = 16
NBRE = 80000
EMB = 128

NW = 32
NPAD = 10240
EPS = 128
DPS = EPS // DEG
C0 = 448
C1 = 192
J0 = C0 // DPS
J1 = C1 // DPS

_f32 = jnp.float32
_i32 = jnp.int32


DEPTH = 4


def _tree_sum(terms):
    while len(terms) > 1:
        nxt = [terms[i] + terms[i + 1] for i in range(0, len(terms) - 1, 2)]
        if len(terms) % 2:
            nxt.append(terms[-1])
        terms = nxt
    return terms[0]


def _sc_body(dev2d, brk_flat, bs_flat, vpre, ne_out, cbs_out,
             dev_v, cbs4, ie4, io4, b04, b14, nb4, rows4, ne4, *sems):
    semc = sems[0:DEPTH]
    semb = sems[DEPTH:2 * DEPTH]
    semv = sems[2 * DEPTH:3 * DEPTH]
    semsc = sems[3 * DEPTH:4 * DEPTH]
    semsn = sems[4 * DEPTH:5 * DEPTH]
    s = lax.axis_index("s")
    c = lax.axis_index("c")
    base = jnp.where(c == 0, s * C0, 16 * C0 + s * C1)
    jcnt = jnp.where(c == 0, J0, J1)
    pltpu.sync_copy(dev2d.at[pl.ds(base // DPS, J0)], dev_v)

    def fire(jj, p):
        @pl.when(jj < jcnt)
        def _():
            @pl.when(jj >= DEPTH)
            def _():
                pltpu.make_async_copy(
                    cbs4.at[p], cbs_out.at[pl.ds(0, EPS)], semsc[p]).wait()
            pltpu.async_copy(bs_flat.at[dev_v.at[jj]], cbs4.at[p], semc[p])
            for v in range(DPS):
                dv2 = dev_v[jj, pl.ds(v * 16, 16)] * 2
                ie4[p, pl.ds(v * 16, 16)] = dv2
                io4[p, pl.ds(v * 16, 16)] = dv2 + 1
            pltpu.async_copy(brk_flat.at[ie4.at[p]], b04.at[p], semb[p])
            pltpu.async_copy(brk_flat.at[io4.at[p]], b14.at[p], semb[p])

    def nb_fire_vpre(jj, p):
        @pl.when(jj < jcnt)
        def _():
            pltpu.make_async_copy(
                brk_flat.at[ie4.at[p]], b04.at[p], semb[p]).wait()
            pltpu.make_async_copy(
                brk_flat.at[io4.at[p]], b14.at[p], semb[p]).wait()
            for v in range(DPS):
                br0 = b04[p, pl.ds(v * 16, 16)]
                br1 = b14[p, pl.ds(v * 16, 16)]
                did = jnp.full((16,), base + jj * DPS + v, _i32)
                nb4[p, pl.ds(v * 16, 16)] = jnp.where(br0 == did, br1, br0)
            pltpu.async_copy(vpre.at[nb4.at[p]], rows4.at[p], semv[p])

    def back(jj, p):
        pltpu.make_async_copy(
            bs_flat.at[dev_v.at[jj]], cbs4.at[p], semc[p]).wait()
        pltpu.async_copy(
            cbs4.at[p], cbs_out.at[pl.ds((base + jj * DPS) * DEG, EPS)],
            semsc[p])
        @pl.when(jj >= DEPTH)
        def _():
            pltpu.make_async_copy(
                ne4.at[p], ne_out.at[pl.ds(base, DPS)], semsn[p]).wait()
        pltpu.make_async_copy(vpre.at[nb4.at[p]], rows4.at[p], semv[p]).wait()

        def red_v(v, carry):
            for eb in range(EMB // 32):
                ta, tb = [], []
                for d in range(DEG):
                    w = rows4[p, v * DEG + d, pl.ds(eb * 16, 16)]
                    ta.append(lax.bitcast_convert_type(w << 16, _f32))
                    tb.append(lax.bitcast_convert_type(
                        w & jnp.int32(-65536), _f32))
                ne4[p, v, pl.ds(eb * 32, 16)] = _tree_sum(ta)
                ne4[p, v, pl.ds(eb * 32 + 16, 16)] = _tree_sum(tb)
            return carry

        lax.fori_loop(0, DPS, red_v, 0)
        pltpu.async_copy(
            ne4.at[p], ne_out.at[pl.ds(base + jj * DPS, DPS)], semsn[p])

    fire(0, 0)
    fire(1, 1)
    fire(2, 2)
    nb_fire_vpre(0, 0)
    nb_fire_vpre(1, 1)

    def body4(i, carry):
        j = 4 * i
        for k in range(4):
            nb_fire_vpre(j + k + 2, (k + 2) % 4)
            back(j + k, k)
            fire(j + k + 3, (k + 3) % 4)
        return carry

    lax.fori_loop(0, jcnt // 4, body4, 0)
    for p in range(DEPTH):
        pltpu.make_async_copy(
            cbs4.at[p], cbs_out.at[pl.ds(0, EPS)], semsc[p]).wait()
        pltpu.make_async_copy(
            ne4.at[p], ne_out.at[pl.ds(0, DPS)], semsn[p]).wait()


@functools.cache
def _sc_gather():
    return pl.kernel(
        _sc_body,
        mesh=plsc.VectorSubcoreMesh(core_axis_name="c", subcore_axis_name="s"),
        out_type=[
            jax.ShapeDtypeStruct((NPAD, EMB), _f32),
            jax.ShapeDtypeStruct((NPAD * DEG,), _f32),
        ],
        scratch_types=[
            pltpu.VMEM((J0, EPS), _i32),
            pltpu.VMEM((DEPTH, EPS), _f32),
            pltpu.VMEM((DEPTH, EPS), _i32),
            pltpu.VMEM((DEPTH, EPS), _i32),
            pltpu.VMEM((DEPTH, EPS), _i32),
            pltpu.VMEM((DEPTH, EPS), _i32),
            pltpu.VMEM((DEPTH, EPS), _i32),
            pltpu.VMEM((DEPTH, EPS, EMB // 2), _i32),
            pltpu.VMEM((DEPTH, DPS, EMB), _f32),
        ] + [pltpu.SemaphoreType.DMA] * (5 * DEPTH),
        compiler_params=pltpu.CompilerParams(use_tc_tiling_on_sc=False),
    )


BLK = 1024


def _tc_body(cbs_ref, ne_ref, ps_ref, W0_ref, W3_ref, W5_ref,
             w1_ref, w2_ref, w4_ref, b0_ref, b1_ref, b2_ref, b3_ref,
             b4_ref, b5_ref, wcb_ref, out_ref):
    cbs = cbs_ref[...]
    w4 = w4_ref[...]
    b4 = b4_ref[...]
    be = jnp.tanh(cbs[:, 0:1] * w4 + b4)
    for d in range(1, DEG):
        be = be + jnp.tanh(cbs[:, d:d + 1] * w4 + b4)
    dn = (((1,), (1,)), ((), ()))
    brk = jnp.tanh(lax.dot_general(be, W3_ref[...], dn,
                                   preferred_element_type=_f32) + b3_ref[...])
    tmp = jnp.sum(cbs, axis=1, keepdims=True)
    tmp_emb = jnp.tanh(tmp * w2_ref[...] + b2_ref[...])
    ps = ps_ref[...]
    pe = 3.0 * tmp_emb
    for p in range(3):
        pe = pe + jnp.tanh(ps[:, p:p + 1] * w1_ref[...] + b1_ref[...])
    pro = jnp.tanh(lax.dot_general(pe, W0_ref[...], dn,
                                   preferred_element_type=_f32) + b0_ref[...])
    nei = jnp.tanh(lax.dot_general(ne_ref[...], W5_ref[...], dn,
                                   preferred_element_type=_f32) + b5_ref[...])
    wcb = wcb_ref[...]
    out_ref[...] = jnp.tanh(pro * wcb[0:1, :] + brk * wcb[1:2, :]
                            + nei * wcb[2:3, :] + wcb[3:4, :])


def _row_spec(width):
    return pl.BlockSpec((BLK, width), lambda i: (i, 0))


def _w_spec(rows, cols):
    return pl.BlockSpec((rows, cols), lambda i: (0, 0))


_tc_dense = pl.pallas_call(
    _tc_body,
    grid=(NPAD // BLK,),
    in_specs=[
        _row_spec(DEG),
        _row_spec(EMB),
        _row_spec(3),
        _w_spec(EMB, EMB),
        _w_spec(EMB, EMB),
        _w_spec(EMB, EMB),
        _w_spec(1, EMB),
        _w_spec(1, EMB),
        _w_spec(1, EMB),
        _w_spec(1, EMB),
        _w_spec(1, EMB),
        _w_spec(1, EMB),
        _w_spec(1, EMB),
        _w_spec(1, EMB),
        _w_spec(1, EMB),
        _w_spec(4, EMB),
    ],
    out_specs=_row_spec(EMB),
    out_shape=jax.ShapeDtypeStruct((NPAD, EMB), _f32),
)


def kernel(V_pre, devices, breakers, protector_sate, breaker_state,
           W0, b0, W1, b1, W2, b2, W3, b3, W4, b4, W5, b5, Wc, bc):
    dev = jnp.pad(devices.astype(_i32), ((0, NPAD - N), (0, 0)))
    dev2d = dev.reshape(NPAD * DEG // EPS, EPS)
    dev2d = jnp.pad(dev2d, ((0, J0), (0, 0)))
    vperm = jnp.transpose(V_pre.reshape(N, EMB // 32, 2, 16),
                          (0, 1, 3, 2)).reshape(N, EMB)
    vpre_b = jax.lax.bitcast_convert_type(
        vperm.astype(jnp.bfloat16).reshape(N, EMB // 2, 2), _i32)
    ne_pad, cbs_flat = _sc_gather()(dev2d, breakers.astype(_i32).reshape(-1),
                                    breaker_state, vpre_b)
    cbs_pad = cbs_flat.reshape(NPAD, DEG)
    ps_pad = jnp.pad(protector_sate, ((0, NPAD - N), (0, 0)))
    row = lambda v: v.reshape(1, EMB)
    wcb = jnp.concatenate([
        jnp.full((1, EMB), Wc[0]), jnp.full((1, EMB), Wc[1]),
        jnp.full((1, EMB), Wc[2]), jnp.full((1, EMB), bc[0]),
    ], axis=0)
    out_pad = _tc_dense(cbs_pad, ne_pad, ps_pad, W0, W3, W5,
                        row(W1[:, 0]), row(W2[:, 0]), row(W4[:, 0]),
                        row(b0), row(b1), row(b2), row(b3), row(b4), row(b5),
                        wcb)
    return out_pad[:N]
